# baseline (speedup 1.0000x reference)
import functools

import jax
import jax.numpy as jnp
from jax import lax
from jax.experimental import pallas as pl
from jax.experimental.pallas import tpu as pltpu
from jax.experimental.pallas import tpu_sc as plsc

_N = 10000
_E = 320000
_D = 128
_T = 4
_STEPS = 4

_NPAD = 10240
_BN = 256
_CHUNK = 128
_NW = 32
_NCH = 80
_PW = _NCH * _CHUNK
_EPAD = _NW * _PW
_NACC = 10104
_RPS = 632
_RPSL = _NACC - 15 * _RPS
_DUMMY = _N


def _types_kernel(h_ref, w_ref, b_ref, y_ref):
    h = h_ref[...]
    for t in range(_T):
        y_ref[t] = (jnp.dot(h, w_ref[t], preferred_element_type=jnp.float32)
                    + b_ref[t][None, :])


def _mh_kernel(h_ref, u_ref, gb_ref, mh_ref):
    mh_ref[...] = (jnp.dot(h_ref[...], u_ref[...],
                           preferred_element_type=jnp.float32)
                   + gb_ref[1][None, :])


def _gru_body(p_ref, h_ref, w_ref, mh_ref, gb_ref):
    msgs = p_ref[0] + p_ref[1]
    mx = (jnp.dot(msgs, w_ref[...], preferred_element_type=jnp.float32)
          + gb_ref[0][None, :])
    h = h_ref[...]
    mh = mh_ref[...]
    z = jax.nn.sigmoid(mx[:, :_D] + mh[:, :_D])
    r = jax.nn.sigmoid(mx[:, _D:2 * _D] + mh[:, _D:2 * _D])
    hh = jnp.tanh(mx[:, 2 * _D:] + r * mh[:, 2 * _D:])
    return z * h + (1.0 - z) * hh


def _gru_kernel(p_ref, h_ref, w_ref, mh_ref, gb_ref, o_ref):
    o_ref[...] = _gru_body(p_ref, h_ref, w_ref, mh_ref, gb_ref)


def _gru_types_kernel(p_ref, h_ref, w_ref, mh_ref, gb_ref, tw_ref, tb_ref,
                      o_ref, y_ref):
    hn = _gru_body(p_ref, h_ref, w_ref, mh_ref, gb_ref)
    o_ref[...] = hn
    for t in range(_T):
        y_ref[t] = (jnp.dot(hn, tw_ref[t], preferred_element_type=jnp.float32)
                    + tb_ref[t][None, :])


_sc_mesh = plsc.VectorSubcoreMesh(core_axis_name="c", subcore_axis_name="s")


_NBUF = 3
_NIDX = 4
_UNROLL = 12


@functools.partial(
    pl.kernel,
    mesh=_sc_mesh,
    out_type=jax.ShapeDtypeStruct((2, _NPAD, _D), jnp.float32),
    scratch_types=[
        pltpu.VMEM((_NIDX, 1, 2 * _CHUNK), jnp.int32),
        pltpu.VMEM((_NBUF, _CHUNK, _D), jnp.float32),
        pltpu.VMEM_SHARED((_NACC, _D), jnp.float32),
        pltpu.SemaphoreType.DMA,
        pltpu.SemaphoreType.DMA,
        pltpu.SemaphoreType.DMA,
        pltpu.SemaphoreType.DMA,
        pltpu.SemaphoreType.DMA,
        pltpu.SemaphoreType.DMA,
        pltpu.SemaphoreType.DMA,
        pltpu.SemaphoreType.DMA,
        pltpu.SemaphoreType.DMA,
        pltpu.SemaphoreType.DMA,
    ],
)
def _edge_pass(y_hbm, idx_hbm, z_hbm, out_hbm, idx_v, rows_v, acc_sh,
               g0, g1, g2, s0, s1, s2, i0, i1, i2, i3):
    c = lax.axis_index("c")
    s = lax.axis_index("s")
    wid = s * 2 + c
    gsem = (g0, g1, g2)
    ssem = (s0, s1, s2)
    isem = (i0, i1, i2, i3)

    def _start_idx(j, sl):
        pltpu.async_copy(idx_hbm.at[wid, j], idx_v.at[sl], isem[sl])

    def _wait_idx(sl):
        pltpu.make_async_copy(idx_hbm.at[wid, 0], idx_v.at[sl],
                              isem[sl]).wait()

    def _gidx(sl):
        return idx_v.at[sl, 0, pl.ds(0, _CHUNK)]

    def _didx(sl):
        return idx_v.at[sl, 0, pl.ds(_CHUNK, _CHUNK)]

    def _wait_rows(sems, b):
        pltpu.make_async_copy(y_hbm.at[pl.ds(0, _CHUNK)],
                              rows_v.at[b], sems[b]).wait()

    def _start_gather(sl, b):
        pltpu.async_copy(y_hbm.at[_gidx(sl)], rows_v.at[b], gsem[b])

    def _start_scatter(b, sl):
        pltpu.async_copy(rows_v.at[b], acc_sh.at[_didx(sl)],
                         ssem[b], add=True)

    def _zero_or_out(write_out):
        def _copy(rows):
            sl_ = pl.ds(s * _RPS, rows)
            if write_out:
                pltpu.sync_copy(acc_sh.at[sl_], out_hbm.at[c, sl_])
            else:
                pltpu.sync_copy(z_hbm.at[sl_], acc_sh.at[sl_])

        pl.when(s < 15)(lambda: _copy(_RPS))
        pl.when(s == 15)(lambda: _copy(_RPSL))

    for k in range(_NIDX):
        _start_idx(k, k)
    _zero_or_out(False)
    for k in range(2):
        _wait_idx(k)
        _start_gather(k, k)
    plsc.subcore_barrier()

    _wait_rows(gsem, 0)
    _start_scatter(0, 0)
    _wait_idx(2)
    _start_gather(2, 2)

    def body(i, carry):
        for k in range(_UNROLL):
            j = 1 + i * _UNROLL + k
            ph = (k + 1) % _UNROLL
            b = ph % _NBUF
            sl = ph % _NIDX
            _wait_rows(gsem, b)
            _start_scatter(b, sl)
            _wait_rows(ssem, (ph - 1) % _NBUF)
            _start_idx(j + 3, (ph + 3) % _NIDX)
            _wait_idx((ph + 2) % _NIDX)
            _start_gather((ph + 2) % _NIDX, (ph + 2) % _NBUF)
        return carry

    lax.fori_loop(0, (_NCH - 8) // _UNROLL, body, 0)

    for j in range(_NCH - 7, _NCH):
        ph = j % _UNROLL
        b = ph % _NBUF
        sl = ph % _NIDX
        _wait_rows(gsem, b)
        _start_scatter(b, sl)
        _wait_rows(ssem, (ph - 1) % _NBUF)
        if j + 3 < _NCH:
            _start_idx(j + 3, (ph + 3) % _NIDX)
        if j + 2 < _NCH:
            _wait_idx((ph + 2) % _NIDX)
            _start_gather((ph + 2) % _NIDX, (ph + 2) % _NBUF)
    _wait_rows(ssem, (_NCH - 1) % _NBUF)

    plsc.subcore_barrier()
    _zero_or_out(True)


def kernel(states, edge_ids, training, type_W, type_b, gru_W, gru_U, gru_b):
    etype = edge_ids[:, 0]
    src = edge_ids[:, 1]
    dst = edge_ids[:, 2]
    pad = _EPAD - _E
    ar = jnp.arange(pad, dtype=jnp.int32)
    gidx = jnp.concatenate([etype * _NPAD + src, ar % (_T * _NPAD)])
    didx = jnp.concatenate([dst, _DUMMY + ar % (_NACC - _DUMMY)])
    idx = jnp.concatenate([gidx.reshape(_NW, _NCH, 1, _CHUNK),
                           didx.reshape(_NW, _NCH, 1, _CHUNK)], axis=3)
    h = jnp.zeros((_NPAD, _D), jnp.float32).at[:_N].set(states)
    zeros_nd = jnp.zeros((_NACC, _D), jnp.float32)

    grid = _NPAD // _BN
    _hs = pl.BlockSpec((_BN, _D), lambda i: (i, 0))
    _ps = pl.BlockSpec((2, _BN, _D), lambda i: (0, i, 0))
    _tws = pl.BlockSpec((_T, _D, _D), lambda i: (0, 0, 0))
    _tbs = pl.BlockSpec((_T, _D), lambda i: (0, 0))
    _us = pl.BlockSpec((_D, 3 * _D), lambda i: (0, 0))
    _gbs = pl.BlockSpec((2, 3 * _D), lambda i: (0, 0))
    _ys = pl.BlockSpec((_T, _BN, _D), lambda i: (0, i, 0))
    _mhs = pl.BlockSpec((_BN, 3 * _D), lambda i: (i, 0))
    _yshape = jax.ShapeDtypeStruct((_T, _NPAD, _D), jnp.float32)
    _hshape = jax.ShapeDtypeStruct((_NPAD, _D), jnp.float32)
    _mhshape = jax.ShapeDtypeStruct((_NPAD, 3 * _D), jnp.float32)

    types_call = pl.pallas_call(
        _types_kernel,
        grid=(grid,),
        in_specs=[_hs, _tws, _tbs],
        out_specs=_ys,
        out_shape=_yshape,
    )
    mh_call = pl.pallas_call(
        _mh_kernel,
        grid=(grid,),
        in_specs=[_hs, _us, _gbs],
        out_specs=_mhs,
        out_shape=_mhshape,
    )
    gru_types_call = pl.pallas_call(
        _gru_types_kernel,
        grid=(grid,),
        in_specs=[_ps, _hs, _us, _mhs, _gbs, _tws, _tbs],
        out_specs=[_hs, _ys],
        out_shape=[_hshape, _yshape],
    )
    gru_call = pl.pallas_call(
        _gru_kernel,
        grid=(grid,),
        in_specs=[_ps, _hs, _us, _mhs, _gbs],
        out_specs=_hs,
        out_shape=_hshape,
    )

    y = types_call(h, type_W, type_b)
    for step in range(_STEPS):
        parts = _edge_pass(y.reshape(_T * _NPAD, _D), idx, zeros_nd)
        mh = mh_call(h, gru_U, gru_b)
        if step < _STEPS - 1:
            h, y = gru_types_call(parts, h, gru_W, mh, gru_b,
                                  type_W, type_b)
        else:
            h = gru_call(parts, h, gru_W, mh, gru_b)
    return h[:_N]


# speedup vs baseline: 1.1260x; 1.1260x over previous
import functools

import jax
import jax.numpy as jnp
from jax import lax
from jax.experimental import pallas as pl
from jax.experimental.pallas import tpu as pltpu
from jax.experimental.pallas import tpu_sc as plsc

_N = 10000
_E = 320000
_D = 128
_T = 4
_STEPS = 4

_NPAD = 10240
_BN = 512
_CHUNK = 128
_NW = 32
_NCH = 80
_PW = _NCH * _CHUNK
_EPAD = _NW * _PW
_NACC = 10104
_RPS = 632
_RPSL = _NACC - 15 * _RPS
_DUMMY = _N


def _types_kernel(h_ref, w_ref, b_ref, y_ref):
    h = h_ref[...]
    for t in range(_T):
        y_ref[t] = (jnp.dot(h, w_ref[t], preferred_element_type=jnp.float32)
                    + b_ref[t][None, :])


def _gru_body(p_ref, h_ref, w_ref, u_ref, gb_ref):
    msgs = p_ref[0] + p_ref[1]
    mx = (jnp.dot(msgs, w_ref[...], preferred_element_type=jnp.float32)
          + gb_ref[0][None, :])
    h = h_ref[...]
    mh = (jnp.dot(h, u_ref[...], preferred_element_type=jnp.float32)
          + gb_ref[1][None, :])
    z = jax.nn.sigmoid(mx[:, :_D] + mh[:, :_D])
    r = jax.nn.sigmoid(mx[:, _D:2 * _D] + mh[:, _D:2 * _D])
    hh = jnp.tanh(mx[:, 2 * _D:] + r * mh[:, 2 * _D:])
    return z * h + (1.0 - z) * hh


def _gru_kernel(p_ref, h_ref, w_ref, u_ref, gb_ref, o_ref):
    o_ref[...] = _gru_body(p_ref, h_ref, w_ref, u_ref, gb_ref)


def _gru_types_kernel(p_ref, h_ref, w_ref, u_ref, gb_ref, tw_ref, tb_ref,
                      o_ref, y_ref):
    hn = _gru_body(p_ref, h_ref, w_ref, u_ref, gb_ref)
    o_ref[...] = hn
    for t in range(_T):
        y_ref[t] = (jnp.dot(hn, tw_ref[t], preferred_element_type=jnp.float32)
                    + tb_ref[t][None, :])


_sc_mesh = plsc.VectorSubcoreMesh(core_axis_name="c", subcore_axis_name="s")


_NBUF = 3
_NIDX = 4
_UNROLL = 12


@functools.partial(
    pl.kernel,
    mesh=_sc_mesh,
    out_type=jax.ShapeDtypeStruct((2, _NPAD, _D), jnp.float32),
    scratch_types=[
        pltpu.VMEM((_NIDX, 1, 2 * _CHUNK), jnp.int32),
        pltpu.VMEM((_NBUF, _CHUNK, _D), jnp.float32),
        pltpu.VMEM_SHARED((_NACC, _D), jnp.float32),
        pltpu.SemaphoreType.DMA,
        pltpu.SemaphoreType.DMA,
        pltpu.SemaphoreType.DMA,
        pltpu.SemaphoreType.DMA,
        pltpu.SemaphoreType.DMA,
        pltpu.SemaphoreType.DMA,
        pltpu.SemaphoreType.DMA,
        pltpu.SemaphoreType.DMA,
        pltpu.SemaphoreType.DMA,
        pltpu.SemaphoreType.DMA,
    ],
)
def _edge_pass(y_hbm, idx_hbm, z_hbm, out_hbm, idx_v, rows_v, acc_sh,
               g0, g1, g2, s0, s1, s2, i0, i1, i2, i3):
    c = lax.axis_index("c")
    s = lax.axis_index("s")
    wid = s * 2 + c
    gsem = (g0, g1, g2)
    ssem = (s0, s1, s2)
    isem = (i0, i1, i2, i3)

    def _start_idx(j, sl):
        pltpu.async_copy(idx_hbm.at[wid, j], idx_v.at[sl], isem[sl])

    def _wait_idx(sl):
        pltpu.make_async_copy(idx_hbm.at[wid, 0], idx_v.at[sl],
                              isem[sl]).wait()

    def _gidx(sl):
        return idx_v.at[sl, 0, pl.ds(0, _CHUNK)]

    def _didx(sl):
        return idx_v.at[sl, 0, pl.ds(_CHUNK, _CHUNK)]

    def _wait_rows(sems, b):
        pltpu.make_async_copy(y_hbm.at[pl.ds(0, _CHUNK)],
                              rows_v.at[b], sems[b]).wait()

    def _start_gather(sl, b):
        pltpu.async_copy(y_hbm.at[_gidx(sl)], rows_v.at[b], gsem[b])

    def _start_scatter(b, sl):
        pltpu.async_copy(rows_v.at[b], acc_sh.at[_didx(sl)],
                         ssem[b], add=True)

    def _zero_or_out(write_out):
        def _copy(rows):
            sl_ = pl.ds(s * _RPS, rows)
            if write_out:
                pltpu.sync_copy(acc_sh.at[sl_], out_hbm.at[c, sl_])
            else:
                pltpu.sync_copy(z_hbm.at[sl_], acc_sh.at[sl_])

        pl.when(s < 15)(lambda: _copy(_RPS))
        pl.when(s == 15)(lambda: _copy(_RPSL))

    for k in range(_NIDX):
        _start_idx(k, k)
    _zero_or_out(False)
    for k in range(2):
        _wait_idx(k)
        _start_gather(k, k)
    plsc.subcore_barrier()

    _wait_rows(gsem, 0)
    _start_scatter(0, 0)
    _wait_idx(2)
    _start_gather(2, 2)

    def body(i, carry):
        for k in range(_UNROLL):
            j = 1 + i * _UNROLL + k
            ph = (k + 1) % _UNROLL
            b = ph % _NBUF
            sl = ph % _NIDX
            _wait_rows(gsem, b)
            _start_scatter(b, sl)
            _wait_rows(ssem, (ph - 1) % _NBUF)
            _start_idx(j + 3, (ph + 3) % _NIDX)
            _wait_idx((ph + 2) % _NIDX)
            _start_gather((ph + 2) % _NIDX, (ph + 2) % _NBUF)
        return carry

    lax.fori_loop(0, (_NCH - 8) // _UNROLL, body, 0)

    for j in range(_NCH - 7, _NCH):
        ph = j % _UNROLL
        b = ph % _NBUF
        sl = ph % _NIDX
        _wait_rows(gsem, b)
        _start_scatter(b, sl)
        _wait_rows(ssem, (ph - 1) % _NBUF)
        if j + 3 < _NCH:
            _start_idx(j + 3, (ph + 3) % _NIDX)
        if j + 2 < _NCH:
            _wait_idx((ph + 2) % _NIDX)
            _start_gather((ph + 2) % _NIDX, (ph + 2) % _NBUF)
    _wait_rows(ssem, (_NCH - 1) % _NBUF)

    plsc.subcore_barrier()
    _zero_or_out(True)


def kernel(states, edge_ids, training, type_W, type_b, gru_W, gru_U, gru_b):
    etype = edge_ids[:, 0]
    src = edge_ids[:, 1]
    dst = edge_ids[:, 2]
    pad = _EPAD - _E
    ar = jnp.arange(pad, dtype=jnp.int32)
    gidx = jnp.concatenate([etype * _NPAD + src, ar % (_T * _NPAD)])
    didx = jnp.concatenate([dst, _DUMMY + ar % (_NACC - _DUMMY)])
    idx = jnp.concatenate([gidx.reshape(_NW, _NCH, 1, _CHUNK),
                           didx.reshape(_NW, _NCH, 1, _CHUNK)], axis=3)
    h = jnp.zeros((_NPAD, _D), jnp.float32).at[:_N].set(states)
    zeros_nd = jnp.zeros((_NACC, _D), jnp.float32)

    grid = _NPAD // _BN
    _hs = pl.BlockSpec((_BN, _D), lambda i: (i, 0))
    _ps = pl.BlockSpec((2, _BN, _D), lambda i: (0, i, 0))
    _tws = pl.BlockSpec((_T, _D, _D), lambda i: (0, 0, 0))
    _tbs = pl.BlockSpec((_T, _D), lambda i: (0, 0))
    _us = pl.BlockSpec((_D, 3 * _D), lambda i: (0, 0))
    _gbs = pl.BlockSpec((2, 3 * _D), lambda i: (0, 0))
    _ys = pl.BlockSpec((_T, _BN, _D), lambda i: (0, i, 0))
    _yshape = jax.ShapeDtypeStruct((_T, _NPAD, _D), jnp.float32)
    _hshape = jax.ShapeDtypeStruct((_NPAD, _D), jnp.float32)

    types_call = pl.pallas_call(
        _types_kernel,
        grid=(grid,),
        in_specs=[_hs, _tws, _tbs],
        out_specs=_ys,
        out_shape=_yshape,
    )
    gru_types_call = pl.pallas_call(
        _gru_types_kernel,
        grid=(grid,),
        in_specs=[_ps, _hs, _us, _us, _gbs, _tws, _tbs],
        out_specs=[_hs, _ys],
        out_shape=[_hshape, _yshape],
    )
    gru_call = pl.pallas_call(
        _gru_kernel,
        grid=(grid,),
        in_specs=[_ps, _hs, _us, _us, _gbs],
        out_specs=_hs,
        out_shape=_hshape,
    )

    y = types_call(h, type_W, type_b)
    for step in range(_STEPS):
        parts = _edge_pass(y.reshape(_T * _NPAD, _D), idx, zeros_nd)
        if step < _STEPS - 1:
            h, y = gru_types_call(parts, h, gru_W, gru_U, gru_b,
                                  type_W, type_b)
        else:
            h = gru_call(parts, h, gru_W, gru_U, gru_b)
    return h[:_N]


# speedup vs baseline: 1.1923x; 1.0589x over previous
import functools

import jax
import jax.numpy as jnp
from jax import lax
from jax.experimental import pallas as pl
from jax.experimental.pallas import tpu as pltpu
from jax.experimental.pallas import tpu_sc as plsc

_N = 10000
_E = 320000
_D = 128
_T = 4
_STEPS = 4

_NPAD = 10240
_BN = 1024
_CHUNK = 128
_NW = 32
_NCH = 80
_PW = _NCH * _CHUNK
_EPAD = _NW * _PW
_NACC = 10104
_RPS = 632
_RPSL = _NACC - 15 * _RPS
_DUMMY = _N


def _types_kernel(h_ref, w_ref, b_ref, y_ref):
    h = h_ref[...]
    for t in range(_T):
        y_ref[t] = (jnp.dot(h, w_ref[t], preferred_element_type=jnp.float32)
                    + b_ref[t][None, :])


def _gru_body(p_ref, h_ref, w_ref, u_ref, gb_ref):
    msgs = p_ref[0] + p_ref[1]
    mx = (jnp.dot(msgs, w_ref[...], preferred_element_type=jnp.float32)
          + gb_ref[0][None, :])
    h = h_ref[...]
    mh = (jnp.dot(h, u_ref[...], preferred_element_type=jnp.float32)
          + gb_ref[1][None, :])
    z = jax.nn.sigmoid(mx[:, :_D] + mh[:, :_D])
    r = jax.nn.sigmoid(mx[:, _D:2 * _D] + mh[:, _D:2 * _D])
    hh = jnp.tanh(mx[:, 2 * _D:] + r * mh[:, 2 * _D:])
    return z * h + (1.0 - z) * hh


def _gru_kernel(p_ref, h_ref, w_ref, u_ref, gb_ref, o_ref):
    o_ref[...] = _gru_body(p_ref, h_ref, w_ref, u_ref, gb_ref)


def _gru_types_kernel(p_ref, h_ref, w_ref, u_ref, gb_ref, tw_ref, tb_ref,
                      o_ref, y_ref):
    hn = _gru_body(p_ref, h_ref, w_ref, u_ref, gb_ref)
    o_ref[...] = hn
    for t in range(_T):
        y_ref[t] = (jnp.dot(hn, tw_ref[t], preferred_element_type=jnp.float32)
                    + tb_ref[t][None, :])


_sc_mesh = plsc.VectorSubcoreMesh(core_axis_name="c", subcore_axis_name="s")


_NBUF = 3
_NIDX = 4
_UNROLL = 12


@functools.partial(
    pl.kernel,
    mesh=_sc_mesh,
    out_type=jax.ShapeDtypeStruct((2, _NPAD, _D), jnp.float32),
    scratch_types=[
        pltpu.VMEM((_NIDX, 1, 2 * _CHUNK), jnp.int32),
        pltpu.VMEM((_NBUF, _CHUNK, _D), jnp.float32),
        pltpu.VMEM_SHARED((_NACC, _D), jnp.float32),
        pltpu.SemaphoreType.DMA,
        pltpu.SemaphoreType.DMA,
        pltpu.SemaphoreType.DMA,
        pltpu.SemaphoreType.DMA,
        pltpu.SemaphoreType.DMA,
        pltpu.SemaphoreType.DMA,
        pltpu.SemaphoreType.DMA,
        pltpu.SemaphoreType.DMA,
        pltpu.SemaphoreType.DMA,
        pltpu.SemaphoreType.DMA,
    ],
)
def _edge_pass(y_hbm, idx_hbm, z_hbm, out_hbm, idx_v, rows_v, acc_sh,
               g0, g1, g2, s0, s1, s2, i0, i1, i2, i3):
    c = lax.axis_index("c")
    s = lax.axis_index("s")
    wid = s * 2 + c
    gsem = (g0, g1, g2)
    ssem = (s0, s1, s2)
    isem = (i0, i1, i2, i3)

    def _start_idx(j, sl):
        pltpu.async_copy(idx_hbm.at[wid, j], idx_v.at[sl], isem[sl])

    def _wait_idx(sl):
        pltpu.make_async_copy(idx_hbm.at[wid, 0], idx_v.at[sl],
                              isem[sl]).wait()

    def _gidx(sl):
        return idx_v.at[sl, 0, pl.ds(0, _CHUNK)]

    def _didx(sl):
        return idx_v.at[sl, 0, pl.ds(_CHUNK, _CHUNK)]

    def _wait_rows(sems, b):
        pltpu.make_async_copy(y_hbm.at[pl.ds(0, _CHUNK)],
                              rows_v.at[b], sems[b]).wait()

    def _start_gather(sl, b):
        pltpu.async_copy(y_hbm.at[_gidx(sl)], rows_v.at[b], gsem[b])

    def _start_scatter(b, sl):
        pltpu.async_copy(rows_v.at[b], acc_sh.at[_didx(sl)],
                         ssem[b], add=True)

    def _zero_or_out(write_out):
        def _copy(rows):
            sl_ = pl.ds(s * _RPS, rows)
            if write_out:
                pltpu.sync_copy(acc_sh.at[sl_], out_hbm.at[c, sl_])
            else:
                pltpu.sync_copy(z_hbm.at[sl_], acc_sh.at[sl_])

        pl.when(s < 15)(lambda: _copy(_RPS))
        pl.when(s == 15)(lambda: _copy(_RPSL))

    for k in range(_NIDX):
        _start_idx(k, k)
    _zero_or_out(False)
    for k in range(2):
        _wait_idx(k)
        _start_gather(k, k)
    plsc.subcore_barrier()

    _wait_rows(gsem, 0)
    _start_scatter(0, 0)
    _wait_idx(2)
    _start_gather(2, 2)

    def body(i, carry):
        for k in range(_UNROLL):
            j = 1 + i * _UNROLL + k
            ph = (k + 1) % _UNROLL
            b = ph % _NBUF
            sl = ph % _NIDX
            _wait_rows(gsem, b)
            _start_scatter(b, sl)
            _wait_rows(ssem, (ph - 1) % _NBUF)
            _start_idx(j + 3, (ph + 3) % _NIDX)
            _wait_idx((ph + 2) % _NIDX)
            _start_gather((ph + 2) % _NIDX, (ph + 2) % _NBUF)
        return carry

    lax.fori_loop(0, (_NCH - 8) // _UNROLL, body, 0)

    for j in range(_NCH - 7, _NCH):
        ph = j % _UNROLL
        b = ph % _NBUF
        sl = ph % _NIDX
        _wait_rows(gsem, b)
        _start_scatter(b, sl)
        _wait_rows(ssem, (ph - 1) % _NBUF)
        if j + 3 < _NCH:
            _start_idx(j + 3, (ph + 3) % _NIDX)
        if j + 2 < _NCH:
            _wait_idx((ph + 2) % _NIDX)
            _start_gather((ph + 2) % _NIDX, (ph + 2) % _NBUF)
    _wait_rows(ssem, (_NCH - 1) % _NBUF)

    plsc.subcore_barrier()
    _zero_or_out(True)


def kernel(states, edge_ids, training, type_W, type_b, gru_W, gru_U, gru_b):
    etype = edge_ids[:, 0]
    src = edge_ids[:, 1]
    dst = edge_ids[:, 2]
    pad = _EPAD - _E
    ar = jnp.arange(pad, dtype=jnp.int32)
    gidx = jnp.concatenate([etype * _NPAD + src, ar % (_T * _NPAD)])
    didx = jnp.concatenate([dst, _DUMMY + ar % (_NACC - _DUMMY)])
    idx = jnp.concatenate([gidx.reshape(_NW, _NCH, 1, _CHUNK),
                           didx.reshape(_NW, _NCH, 1, _CHUNK)], axis=3)
    h = jnp.zeros((_NPAD, _D), jnp.float32).at[:_N].set(states)
    zeros_nd = jnp.zeros((_NACC, _D), jnp.float32)

    grid = _NPAD // _BN
    _hs = pl.BlockSpec((_BN, _D), lambda i: (i, 0))
    _ps = pl.BlockSpec((2, _BN, _D), lambda i: (0, i, 0))
    _tws = pl.BlockSpec((_T, _D, _D), lambda i: (0, 0, 0))
    _tbs = pl.BlockSpec((_T, _D), lambda i: (0, 0))
    _us = pl.BlockSpec((_D, 3 * _D), lambda i: (0, 0))
    _gbs = pl.BlockSpec((2, 3 * _D), lambda i: (0, 0))
    _ys = pl.BlockSpec((_T, _BN, _D), lambda i: (0, i, 0))
    _yshape = jax.ShapeDtypeStruct((_T, _NPAD, _D), jnp.float32)
    _hshape = jax.ShapeDtypeStruct((_NPAD, _D), jnp.float32)

    types_call = pl.pallas_call(
        _types_kernel,
        grid=(grid,),
        in_specs=[_hs, _tws, _tbs],
        out_specs=_ys,
        out_shape=_yshape,
    )
    gru_types_call = pl.pallas_call(
        _gru_types_kernel,
        grid=(grid,),
        in_specs=[_ps, _hs, _us, _us, _gbs, _tws, _tbs],
        out_specs=[_hs, _ys],
        out_shape=[_hshape, _yshape],
    )
    gru_call = pl.pallas_call(
        _gru_kernel,
        grid=(grid,),
        in_specs=[_ps, _hs, _us, _us, _gbs],
        out_specs=_hs,
        out_shape=_hshape,
    )

    y = types_call(h, type_W, type_b)
    for step in range(_STEPS):
        parts = _edge_pass(y.reshape(_T * _NPAD, _D), idx, zeros_nd)
        if step < _STEPS - 1:
            h, y = gru_types_call(parts, h, gru_W, gru_U, gru_b,
                                  type_W, type_b)
        else:
            h = gru_call(parts, h, gru_W, gru_U, gru_b)
    return h[:_N]


# speedup vs baseline: 1.2177x; 1.0213x over previous
import functools

import jax
import jax.numpy as jnp
from jax import lax
from jax.experimental import pallas as pl
from jax.experimental.pallas import tpu as pltpu
from jax.experimental.pallas import tpu_sc as plsc

_N = 10000
_E = 320000
_D = 128
_T = 4
_STEPS = 4

_NPAD = 10240
_BN = 2048
_CHUNK = 128
_NW = 32
_NCH = 80
_PW = _NCH * _CHUNK
_EPAD = _NW * _PW
_NACC = 10104
_RPS = 632
_RPSL = _NACC - 15 * _RPS
_DUMMY = _N


def _types_kernel(h_ref, w_ref, b_ref, y_ref):
    h = h_ref[...]
    for t in range(_T):
        y_ref[t] = (jnp.dot(h, w_ref[t], preferred_element_type=jnp.float32)
                    + b_ref[t][None, :])


def _gru_body(p_ref, h_ref, w_ref, u_ref, gb_ref):
    msgs = p_ref[0] + p_ref[1]
    mx = (jnp.dot(msgs, w_ref[...], preferred_element_type=jnp.float32)
          + gb_ref[0][None, :])
    h = h_ref[...]
    mh = (jnp.dot(h, u_ref[...], preferred_element_type=jnp.float32)
          + gb_ref[1][None, :])
    z = jax.nn.sigmoid(mx[:, :_D] + mh[:, :_D])
    r = jax.nn.sigmoid(mx[:, _D:2 * _D] + mh[:, _D:2 * _D])
    hh = jnp.tanh(mx[:, 2 * _D:] + r * mh[:, 2 * _D:])
    return z * h + (1.0 - z) * hh


def _gru_kernel(p_ref, h_ref, w_ref, u_ref, gb_ref, o_ref):
    o_ref[...] = _gru_body(p_ref, h_ref, w_ref, u_ref, gb_ref)


def _gru_types_kernel(p_ref, h_ref, w_ref, u_ref, gb_ref, tw_ref, tb_ref,
                      o_ref, y_ref):
    hn = _gru_body(p_ref, h_ref, w_ref, u_ref, gb_ref)
    o_ref[...] = hn
    for t in range(_T):
        y_ref[t] = (jnp.dot(hn, tw_ref[t], preferred_element_type=jnp.float32)
                    + tb_ref[t][None, :])


_sc_mesh = plsc.VectorSubcoreMesh(core_axis_name="c", subcore_axis_name="s")


_NBUF = 3
_NIDX = 4
_UNROLL = 12


@functools.partial(
    pl.kernel,
    mesh=_sc_mesh,
    out_type=jax.ShapeDtypeStruct((2, _NPAD, _D), jnp.float32),
    scratch_types=[
        pltpu.VMEM((_NIDX, 1, 2 * _CHUNK), jnp.int32),
        pltpu.VMEM((_NBUF, _CHUNK, _D), jnp.float32),
        pltpu.VMEM_SHARED((_NACC, _D), jnp.float32),
        pltpu.SemaphoreType.DMA,
        pltpu.SemaphoreType.DMA,
        pltpu.SemaphoreType.DMA,
        pltpu.SemaphoreType.DMA,
        pltpu.SemaphoreType.DMA,
        pltpu.SemaphoreType.DMA,
        pltpu.SemaphoreType.DMA,
        pltpu.SemaphoreType.DMA,
        pltpu.SemaphoreType.DMA,
        pltpu.SemaphoreType.DMA,
    ],
)
def _edge_pass(y_hbm, idx_hbm, z_hbm, out_hbm, idx_v, rows_v, acc_sh,
               g0, g1, g2, s0, s1, s2, i0, i1, i2, i3):
    c = lax.axis_index("c")
    s = lax.axis_index("s")
    wid = s * 2 + c
    gsem = (g0, g1, g2)
    ssem = (s0, s1, s2)
    isem = (i0, i1, i2, i3)

    def _start_idx(j, sl):
        pltpu.async_copy(idx_hbm.at[wid, j], idx_v.at[sl], isem[sl])

    def _wait_idx(sl):
        pltpu.make_async_copy(idx_hbm.at[wid, 0], idx_v.at[sl],
                              isem[sl]).wait()

    def _gidx(sl):
        return idx_v.at[sl, 0, pl.ds(0, _CHUNK)]

    def _didx(sl):
        return idx_v.at[sl, 0, pl.ds(_CHUNK, _CHUNK)]

    def _wait_rows(sems, b):
        pltpu.make_async_copy(y_hbm.at[pl.ds(0, _CHUNK)],
                              rows_v.at[b], sems[b]).wait()

    def _start_gather(sl, b):
        pltpu.async_copy(y_hbm.at[_gidx(sl)], rows_v.at[b], gsem[b])

    def _start_scatter(b, sl):
        pltpu.async_copy(rows_v.at[b], acc_sh.at[_didx(sl)],
                         ssem[b], add=True)

    def _zero_or_out(write_out):
        def _copy(rows):
            sl_ = pl.ds(s * _RPS, rows)
            if write_out:
                pltpu.sync_copy(acc_sh.at[sl_], out_hbm.at[c, sl_])
            else:
                pltpu.sync_copy(z_hbm.at[sl_], acc_sh.at[sl_])

        pl.when(s < 15)(lambda: _copy(_RPS))
        pl.when(s == 15)(lambda: _copy(_RPSL))

    for k in range(_NIDX):
        _start_idx(k, k)
    _zero_or_out(False)
    for k in range(2):
        _wait_idx(k)
        _start_gather(k, k)
    plsc.subcore_barrier()

    _wait_rows(gsem, 0)
    _start_scatter(0, 0)
    _wait_idx(2)
    _start_gather(2, 2)

    def body(i, carry):
        for k in range(_UNROLL):
            j = 1 + i * _UNROLL + k
            ph = (k + 1) % _UNROLL
            b = ph % _NBUF
            sl = ph % _NIDX
            _wait_rows(gsem, b)
            _start_scatter(b, sl)
            _wait_rows(ssem, (ph - 1) % _NBUF)
            _start_idx(j + 3, (ph + 3) % _NIDX)
            _wait_idx((ph + 2) % _NIDX)
            _start_gather((ph + 2) % _NIDX, (ph + 2) % _NBUF)
        return carry

    lax.fori_loop(0, (_NCH - 8) // _UNROLL, body, 0)

    for j in range(_NCH - 7, _NCH):
        ph = j % _UNROLL
        b = ph % _NBUF
        sl = ph % _NIDX
        _wait_rows(gsem, b)
        _start_scatter(b, sl)
        _wait_rows(ssem, (ph - 1) % _NBUF)
        if j + 3 < _NCH:
            _start_idx(j + 3, (ph + 3) % _NIDX)
        if j + 2 < _NCH:
            _wait_idx((ph + 2) % _NIDX)
            _start_gather((ph + 2) % _NIDX, (ph + 2) % _NBUF)
    _wait_rows(ssem, (_NCH - 1) % _NBUF)

    plsc.subcore_barrier()
    _zero_or_out(True)


def kernel(states, edge_ids, training, type_W, type_b, gru_W, gru_U, gru_b):
    etype = edge_ids[:, 0]
    src = edge_ids[:, 1]
    dst = edge_ids[:, 2]
    pad = _EPAD - _E
    ar = jnp.arange(pad, dtype=jnp.int32)
    gidx = jnp.concatenate([etype * _NPAD + src, ar % (_T * _NPAD)])
    didx = jnp.concatenate([dst, _DUMMY + ar % (_NACC - _DUMMY)])
    idx = jnp.concatenate([gidx.reshape(_NW, _NCH, 1, _CHUNK),
                           didx.reshape(_NW, _NCH, 1, _CHUNK)], axis=3)
    h = jnp.zeros((_NPAD, _D), jnp.float32).at[:_N].set(states)
    zeros_nd = jnp.zeros((_NACC, _D), jnp.float32)

    grid = _NPAD // _BN
    _hs = pl.BlockSpec((_BN, _D), lambda i: (i, 0))
    _ps = pl.BlockSpec((2, _BN, _D), lambda i: (0, i, 0))
    _tws = pl.BlockSpec((_T, _D, _D), lambda i: (0, 0, 0))
    _tbs = pl.BlockSpec((_T, _D), lambda i: (0, 0))
    _us = pl.BlockSpec((_D, 3 * _D), lambda i: (0, 0))
    _gbs = pl.BlockSpec((2, 3 * _D), lambda i: (0, 0))
    _ys = pl.BlockSpec((_T, _BN, _D), lambda i: (0, i, 0))
    _yshape = jax.ShapeDtypeStruct((_T, _NPAD, _D), jnp.float32)
    _hshape = jax.ShapeDtypeStruct((_NPAD, _D), jnp.float32)

    types_call = pl.pallas_call(
        _types_kernel,
        grid=(grid,),
        in_specs=[_hs, _tws, _tbs],
        out_specs=_ys,
        out_shape=_yshape,
    )
    gru_types_call = pl.pallas_call(
        _gru_types_kernel,
        grid=(grid,),
        in_specs=[_ps, _hs, _us, _us, _gbs, _tws, _tbs],
        out_specs=[_hs, _ys],
        out_shape=[_hshape, _yshape],
    )
    gru_call = pl.pallas_call(
        _gru_kernel,
        grid=(grid,),
        in_specs=[_ps, _hs, _us, _us, _gbs],
        out_specs=_hs,
        out_shape=_hshape,
    )

    y = types_call(h, type_W, type_b)
    for step in range(_STEPS):
        parts = _edge_pass(y.reshape(_T * _NPAD, _D), idx, zeros_nd)
        if step < _STEPS - 1:
            h, y = gru_types_call(parts, h, gru_W, gru_U, gru_b,
                                  type_W, type_b)
        else:
            h = gru_call(parts, h, gru_W, gru_U, gru_b)
    return h[:_N]


# speedup vs baseline: 1.2238x; 1.0050x over previous
import functools

import jax
import jax.numpy as jnp
from jax import lax
from jax.experimental import pallas as pl
from jax.experimental.pallas import tpu as pltpu
from jax.experimental.pallas import tpu_sc as plsc

_N = 10000
_E = 320000
_D = 128
_T = 4
_STEPS = 4

_NPAD = 10240
_BN = 2560
_CHUNK = 128
_NW = 32
_NCH = 80
_PW = _NCH * _CHUNK
_EPAD = _NW * _PW
_NACC = 10104
_RPS = 632
_RPSL = _NACC - 15 * _RPS
_DUMMY = _N


def _types_kernel(h_ref, w_ref, b_ref, y_ref):
    h = h_ref[...]
    for t in range(_T):
        y_ref[t] = (jnp.dot(h, w_ref[t], preferred_element_type=jnp.float32)
                    + b_ref[t][None, :])


def _gru_body(p_ref, h_ref, w_ref, u_ref, gb_ref):
    msgs = p_ref[0] + p_ref[1]
    mx = (jnp.dot(msgs, w_ref[...], preferred_element_type=jnp.float32)
          + gb_ref[0][None, :])
    h = h_ref[...]
    mh = (jnp.dot(h, u_ref[...], preferred_element_type=jnp.float32)
          + gb_ref[1][None, :])
    z = jax.nn.sigmoid(mx[:, :_D] + mh[:, :_D])
    r = jax.nn.sigmoid(mx[:, _D:2 * _D] + mh[:, _D:2 * _D])
    hh = jnp.tanh(mx[:, 2 * _D:] + r * mh[:, 2 * _D:])
    return z * h + (1.0 - z) * hh


def _gru_kernel(p_ref, h_ref, w_ref, u_ref, gb_ref, o_ref):
    o_ref[...] = _gru_body(p_ref, h_ref, w_ref, u_ref, gb_ref)


def _gru_types_kernel(p_ref, h_ref, w_ref, u_ref, gb_ref, tw_ref, tb_ref,
                      o_ref, y_ref):
    hn = _gru_body(p_ref, h_ref, w_ref, u_ref, gb_ref)
    o_ref[...] = hn
    for t in range(_T):
        y_ref[t] = (jnp.dot(hn, tw_ref[t], preferred_element_type=jnp.float32)
                    + tb_ref[t][None, :])


_sc_mesh = plsc.VectorSubcoreMesh(core_axis_name="c", subcore_axis_name="s")


_NBUF = 3
_NIDX = 4
_UNROLL = 12


@functools.partial(
    pl.kernel,
    mesh=_sc_mesh,
    out_type=jax.ShapeDtypeStruct((2, _NPAD, _D), jnp.float32),
    scratch_types=[
        pltpu.VMEM((_NIDX, 1, 2 * _CHUNK), jnp.int32),
        pltpu.VMEM((_NBUF, _CHUNK, _D), jnp.float32),
        pltpu.VMEM_SHARED((_NACC, _D), jnp.float32),
        pltpu.SemaphoreType.DMA,
        pltpu.SemaphoreType.DMA,
        pltpu.SemaphoreType.DMA,
        pltpu.SemaphoreType.DMA,
        pltpu.SemaphoreType.DMA,
        pltpu.SemaphoreType.DMA,
        pltpu.SemaphoreType.DMA,
        pltpu.SemaphoreType.DMA,
        pltpu.SemaphoreType.DMA,
        pltpu.SemaphoreType.DMA,
    ],
)
def _edge_pass(y_hbm, idx_hbm, z_hbm, out_hbm, idx_v, rows_v, acc_sh,
               g0, g1, g2, s0, s1, s2, i0, i1, i2, i3):
    c = lax.axis_index("c")
    s = lax.axis_index("s")
    wid = s * 2 + c
    gsem = (g0, g1, g2)
    ssem = (s0, s1, s2)
    isem = (i0, i1, i2, i3)

    def _start_idx(j, sl):
        pltpu.async_copy(idx_hbm.at[wid, j], idx_v.at[sl], isem[sl])

    def _wait_idx(sl):
        pltpu.make_async_copy(idx_hbm.at[wid, 0], idx_v.at[sl],
                              isem[sl]).wait()

    def _gidx(sl):
        return idx_v.at[sl, 0, pl.ds(0, _CHUNK)]

    def _didx(sl):
        return idx_v.at[sl, 0, pl.ds(_CHUNK, _CHUNK)]

    def _wait_rows(sems, b):
        pltpu.make_async_copy(y_hbm.at[pl.ds(0, _CHUNK)],
                              rows_v.at[b], sems[b]).wait()

    def _start_gather(sl, b):
        pltpu.async_copy(y_hbm.at[_gidx(sl)], rows_v.at[b], gsem[b])

    def _start_scatter(b, sl):
        pltpu.async_copy(rows_v.at[b], acc_sh.at[_didx(sl)],
                         ssem[b], add=True)

    def _zero_or_out(write_out):
        def _copy(rows):
            sl_ = pl.ds(s * _RPS, rows)
            if write_out:
                pltpu.sync_copy(acc_sh.at[sl_], out_hbm.at[c, sl_])
            else:
                pltpu.sync_copy(z_hbm.at[sl_], acc_sh.at[sl_])

        pl.when(s < 15)(lambda: _copy(_RPS))
        pl.when(s == 15)(lambda: _copy(_RPSL))

    for k in range(_NIDX):
        _start_idx(k, k)
    _zero_or_out(False)
    for k in range(2):
        _wait_idx(k)
        _start_gather(k, k)
    plsc.subcore_barrier()

    _wait_rows(gsem, 0)
    _start_scatter(0, 0)
    _wait_idx(2)
    _start_gather(2, 2)

    def body(i, carry):
        for k in range(_UNROLL):
            j = 1 + i * _UNROLL + k
            ph = (k + 1) % _UNROLL
            b = ph % _NBUF
            sl = ph % _NIDX
            _wait_rows(gsem, b)
            _start_scatter(b, sl)
            _wait_rows(ssem, (ph - 1) % _NBUF)
            _start_idx(j + 3, (ph + 3) % _NIDX)
            _wait_idx((ph + 2) % _NIDX)
            _start_gather((ph + 2) % _NIDX, (ph + 2) % _NBUF)
        return carry

    lax.fori_loop(0, (_NCH - 8) // _UNROLL, body, 0)

    for j in range(_NCH - 7, _NCH):
        ph = j % _UNROLL
        b = ph % _NBUF
        sl = ph % _NIDX
        _wait_rows(gsem, b)
        _start_scatter(b, sl)
        _wait_rows(ssem, (ph - 1) % _NBUF)
        if j + 3 < _NCH:
            _start_idx(j + 3, (ph + 3) % _NIDX)
        if j + 2 < _NCH:
            _wait_idx((ph + 2) % _NIDX)
            _start_gather((ph + 2) % _NIDX, (ph + 2) % _NBUF)
    _wait_rows(ssem, (_NCH - 1) % _NBUF)

    plsc.subcore_barrier()
    _zero_or_out(True)


def kernel(states, edge_ids, training, type_W, type_b, gru_W, gru_U, gru_b):
    etype = edge_ids[:, 0]
    src = edge_ids[:, 1]
    dst = edge_ids[:, 2]
    pad = _EPAD - _E
    ar = jnp.arange(pad, dtype=jnp.int32)
    gidx = jnp.concatenate([etype * _NPAD + src, ar % (_T * _NPAD)])
    didx = jnp.concatenate([dst, _DUMMY + ar % (_NACC - _DUMMY)])
    idx = jnp.concatenate([gidx.reshape(_NW, _NCH, 1, _CHUNK),
                           didx.reshape(_NW, _NCH, 1, _CHUNK)], axis=3)
    h = jnp.zeros((_NPAD, _D), jnp.float32).at[:_N].set(states)
    zeros_nd = jnp.zeros((_NACC, _D), jnp.float32)

    grid = _NPAD // _BN
    _hs = pl.BlockSpec((_BN, _D), lambda i: (i, 0))
    _ps = pl.BlockSpec((2, _BN, _D), lambda i: (0, i, 0))
    _tws = pl.BlockSpec((_T, _D, _D), lambda i: (0, 0, 0))
    _tbs = pl.BlockSpec((_T, _D), lambda i: (0, 0))
    _us = pl.BlockSpec((_D, 3 * _D), lambda i: (0, 0))
    _gbs = pl.BlockSpec((2, 3 * _D), lambda i: (0, 0))
    _ys = pl.BlockSpec((_T, _BN, _D), lambda i: (0, i, 0))
    _yshape = jax.ShapeDtypeStruct((_T, _NPAD, _D), jnp.float32)
    _hshape = jax.ShapeDtypeStruct((_NPAD, _D), jnp.float32)

    types_call = pl.pallas_call(
        _types_kernel,
        grid=(grid,),
        in_specs=[_hs, _tws, _tbs],
        out_specs=_ys,
        out_shape=_yshape,
    )
    gru_types_call = pl.pallas_call(
        _gru_types_kernel,
        grid=(grid,),
        in_specs=[_ps, _hs, _us, _us, _gbs, _tws, _tbs],
        out_specs=[_hs, _ys],
        out_shape=[_hshape, _yshape],
    )
    gru_call = pl.pallas_call(
        _gru_kernel,
        grid=(grid,),
        in_specs=[_ps, _hs, _us, _us, _gbs],
        out_specs=_hs,
        out_shape=_hshape,
    )

    y = types_call(h, type_W, type_b)
    for step in range(_STEPS):
        parts = _edge_pass(y.reshape(_T * _NPAD, _D), idx, zeros_nd)
        if step < _STEPS - 1:
            h, y = gru_types_call(parts, h, gru_W, gru_U, gru_b,
                                  type_W, type_b)
        else:
            h = gru_call(parts, h, gru_W, gru_U, gru_b)
    return h[:_N]
